# baseline (speedup 1.0000x reference)
import functools

import jax
import jax.numpy as jnp
from jax import lax
from jax.experimental import pallas as pl
from jax.experimental.pallas import tpu as pltpu
from jax.experimental.pallas import tpu_sc as plsc

NC = 2
NS = 16
NW = NC * NS

EMB = 64
BLK = 128
L = 16


def _transpose_rows_to_tiles(src, dst, rows, cols):
    lanes = lax.iota(jnp.int32, L)
    for c in range(cols):
        for r0 in range(0, rows, L):
            v = plsc.load_gather(src, [lanes + r0, jnp.full((L,), c, jnp.int32)])
            dst[c // 8, c % 8, pl.ds(r0, L)] = v


def _emb_body(idx_hbm, table_hbm, out_hbm, idxa, idxt, g0, g1, t0, t1,
              gsem0, gsem1, osem0, osem1, *, seq):
    wid = lax.axis_index("s") * NC + lax.axis_index("c")
    base = wid * BLK

    pltpu.sync_copy(idx_hbm.at[pl.ds(base, BLK)], idxa)
    lanes = lax.iota(jnp.int32, L)
    for s in range(seq):
        for r0 in range(0, BLK, L):
            v = plsc.load_gather(idxa, [lanes + r0, jnp.full((L,), s, jnp.int32)])
            idxt[s, pl.ds(r0, L)] = v

    g = (g0, g1)
    t = (t0, t1)
    gsems = (gsem0, gsem1)
    osems = (osem0, osem1)

    def gather_descr(s, b):
        return pltpu.make_async_copy(table_hbm.at[idxt.at[s]], g[b], gsems[b])

    def out_descr(s, b):
        return pltpu.make_async_copy(t[b], out_hbm.at[s, :, wid], osems[b])

    gather_descr(0, 0).start()
    gather_descr(1, 1).start()

    def loop_body(i, carry):
        del carry
        for b in range(2):
            s = 2 * i + b

            @pl.when(s >= 2)
            def _():
                out_descr(s - 2, b).wait()

            gather_descr(s, b).wait()
            _transpose_rows_to_tiles(g[b], t[b], BLK, EMB)
            out_descr(s, b).start()

            @pl.when(s + 2 < seq)
            def _():
                gather_descr(s + 2, b).start()

        return 0

    lax.fori_loop(0, seq // 2, loop_body, 0)

    for b in range(2):
        out_descr(seq - 2 + b, b).wait()


def _emb_lookup(idx_bm, table, batch, seq):
    mesh = plsc.VectorSubcoreMesh(core_axis_name="c", subcore_axis_name="s")
    body = functools.partial(_emb_body, seq=seq)
    return pl.kernel(
        body,
        out_type=jax.ShapeDtypeStruct((seq, EMB // 8, batch // BLK, 8, BLK),
                                      jnp.float32),
        mesh=mesh,
        compiler_params=pltpu.CompilerParams(use_tc_tiling_on_sc=False,
                                             needs_layout_passes=False),
        scratch_types=[
            pltpu.VMEM((BLK, seq), jnp.int32),
            pltpu.VMEM((seq, BLK), jnp.int32),
            pltpu.VMEM((BLK, EMB), jnp.float32),
            pltpu.VMEM((BLK, EMB), jnp.float32),
            pltpu.VMEM((EMB // 8, 8, BLK), jnp.float32),
            pltpu.VMEM((EMB // 8, 8, BLK), jnp.float32),
            pltpu.SemaphoreType.DMA,
            pltpu.SemaphoreType.DMA,
            pltpu.SemaphoreType.DMA,
            pltpu.SemaphoreType.DMA,
        ],
    )(idx_bm, table)


def kernel(x, table):
    batch, seq = x.shape
    idx_bm = x.astype(jnp.int32)
    out5 = _emb_lookup(idx_bm, table, batch, seq)
    out = jnp.transpose(out5, (2, 4, 0, 1, 3))
    return jnp.reshape(out, (batch, seq, EMB))


# speedup vs baseline: 1.8127x; 1.8127x over previous
import functools

import jax
import jax.numpy as jnp
from jax import lax
from jax.experimental import pallas as pl
from jax.experimental.pallas import tpu as pltpu
from jax.experimental.pallas import tpu_sc as plsc

NC = 2
NS = 16
NW = NC * NS

EMB = 64
BLK = 128
L = 16


TP = 129


def _transpose_rows_to_tiles(src, dst, rows, cols):
    lanes = lax.iota(jnp.int32, L)
    for c0 in range(0, cols, L):
        hi = (c0 + lanes) // 8
        lo = (c0 + lanes) % 8
        for r in range(rows):
            v = src[r, pl.ds(c0, L)]
            plsc.store_scatter(dst, [hi, lo, jnp.full((L,), r, jnp.int32)], v)


def _emb_body(idx_hbm, table_hbm, out_hbm, idxa, idxt, g0, g1, t0, t1,
              gsem0, gsem1, osem0, osem1, *, seq):
    wid = lax.axis_index("s") * NC + lax.axis_index("c")
    base = wid * BLK

    pltpu.sync_copy(idx_hbm.at[pl.ds(base, BLK)], idxa)
    lanes = lax.iota(jnp.int32, L)
    for s in range(seq):
        for r0 in range(0, BLK, L):
            v = plsc.load_gather(idxa, [lanes + r0, jnp.full((L,), s, jnp.int32)])
            idxt[s, pl.ds(r0, L)] = v

    g = (g0, g1)
    t = (t0, t1)
    gsems = (gsem0, gsem1)
    osems = (osem0, osem1)

    def gather_descr(s, b):
        return pltpu.make_async_copy(table_hbm.at[idxt.at[s]], g[b], gsems[b])

    def out_descr(s, b):
        return pltpu.make_async_copy(t[b].at[:, :, pl.ds(0, BLK)],
                                     out_hbm.at[s, :, wid], osems[b])

    gather_descr(0, 0).start()
    gather_descr(1, 1).start()

    def loop_body(i, carry):
        del carry
        for b in range(2):
            s = 2 * i + b

            @pl.when(s >= 2)
            def _():
                out_descr(s - 2, b).wait()

            gather_descr(s, b).wait()
            _transpose_rows_to_tiles(g[b], t[b], BLK, EMB)
            out_descr(s, b).start()

            @pl.when(s + 2 < seq)
            def _():
                gather_descr(s + 2, b).start()

        return 0

    lax.fori_loop(0, seq // 2, loop_body, 0)

    for b in range(2):
        out_descr(seq - 2 + b, b).wait()


def _emb_lookup(idx_bm, table, batch, seq):
    mesh = plsc.VectorSubcoreMesh(core_axis_name="c", subcore_axis_name="s")
    body = functools.partial(_emb_body, seq=seq)
    return pl.kernel(
        body,
        out_type=jax.ShapeDtypeStruct((seq, EMB // 8, batch // BLK, 8, BLK),
                                      jnp.float32),
        mesh=mesh,
        compiler_params=pltpu.CompilerParams(use_tc_tiling_on_sc=False,
                                             needs_layout_passes=False),
        scratch_types=[
            pltpu.VMEM((BLK, seq), jnp.int32),
            pltpu.VMEM((seq, BLK), jnp.int32),
            pltpu.VMEM((BLK, EMB), jnp.float32),
            pltpu.VMEM((BLK, EMB), jnp.float32),
            pltpu.VMEM((EMB // 8, 8, TP), jnp.float32),
            pltpu.VMEM((EMB // 8, 8, TP), jnp.float32),
            pltpu.SemaphoreType.DMA,
            pltpu.SemaphoreType.DMA,
            pltpu.SemaphoreType.DMA,
            pltpu.SemaphoreType.DMA,
        ],
    )(idx_bm, table)


def kernel(x, table):
    batch, seq = x.shape
    idx_bm = x.astype(jnp.int32)
    out5 = _emb_lookup(idx_bm, table, batch, seq)
    out = jnp.transpose(out5, (2, 4, 0, 1, 3))
    return jnp.reshape(out, (batch, seq, EMB))


# speedup vs baseline: 1.9049x; 1.0509x over previous
import functools

import jax
import jax.numpy as jnp
from jax import lax
from jax.experimental import pallas as pl
from jax.experimental.pallas import tpu as pltpu
from jax.experimental.pallas import tpu_sc as plsc

NC = 2
NS = 16
NW = NC * NS

EMB = 64
BLK = 128
L = 16


TP = 129


def _transpose_rows_to_tiles(src, dst, rows, cols):
    lanes = lax.iota(jnp.int32, L)
    G = 8
    for c0 in range(0, cols, L):
        hi = (c0 + lanes) // 8
        lo = (c0 + lanes) % 8

        def store(r, v):
            plsc.store_scatter(dst, [hi, lo, jnp.full((L,), r, jnp.int32)], v)

        vs = [src[r, pl.ds(c0, L)] for r in range(G)]
        for r0 in range(G, rows, G):
            for k in range(G):
                store(r0 - G + k, vs[k])
                vs[k] = src[r0 + k, pl.ds(c0, L)]
        for k in range(G):
            store(rows - G + k, vs[k])


def _emb_body(idx_hbm, table_hbm, out_hbm, idxa, idxt, g0, g1, t0, t1,
              gsem0, gsem1, osem0, osem1, *, seq):
    wid = lax.axis_index("s") * NC + lax.axis_index("c")
    base = wid * BLK

    pltpu.sync_copy(idx_hbm.at[pl.ds(base, BLK)], idxa)
    lanes = lax.iota(jnp.int32, L)
    for s in range(seq):
        for r0 in range(0, BLK, L):
            v = plsc.load_gather(idxa, [lanes + r0, jnp.full((L,), s, jnp.int32)])
            idxt[s, pl.ds(r0, L)] = v

    g = (g0, g1)
    t = (t0, t1)
    gsems = (gsem0, gsem1)
    osems = (osem0, osem1)

    def gather_descr(s, b):
        return pltpu.make_async_copy(table_hbm.at[idxt.at[s]], g[b], gsems[b])

    def out_descr(s, b):
        return pltpu.make_async_copy(t[b].at[:, :, pl.ds(0, BLK)],
                                     out_hbm.at[s, :, wid], osems[b])

    gather_descr(0, 0).start()
    gather_descr(1, 1).start()

    def loop_body(i, carry):
        del carry
        for b in range(2):
            s = 2 * i + b

            @pl.when(s >= 2)
            def _():
                out_descr(s - 2, b).wait()

            gather_descr(s, b).wait()
            _transpose_rows_to_tiles(g[b], t[b], BLK, EMB)
            out_descr(s, b).start()

            @pl.when(s + 2 < seq)
            def _():
                gather_descr(s + 2, b).start()

        return 0

    lax.fori_loop(0, seq // 2, loop_body, 0)

    for b in range(2):
        out_descr(seq - 2 + b, b).wait()


def _emb_lookup(idx_bm, table, batch, seq):
    mesh = plsc.VectorSubcoreMesh(core_axis_name="c", subcore_axis_name="s")
    body = functools.partial(_emb_body, seq=seq)
    return pl.kernel(
        body,
        out_type=jax.ShapeDtypeStruct((seq, EMB // 8, batch // BLK, 8, BLK),
                                      jnp.float32),
        mesh=mesh,
        compiler_params=pltpu.CompilerParams(use_tc_tiling_on_sc=False,
                                             needs_layout_passes=False),
        scratch_types=[
            pltpu.VMEM((BLK, seq), jnp.int32),
            pltpu.VMEM((seq, BLK), jnp.int32),
            pltpu.VMEM((BLK, EMB), jnp.float32),
            pltpu.VMEM((BLK, EMB), jnp.float32),
            pltpu.VMEM((EMB // 8, 8, TP), jnp.float32),
            pltpu.VMEM((EMB // 8, 8, TP), jnp.float32),
            pltpu.SemaphoreType.DMA,
            pltpu.SemaphoreType.DMA,
            pltpu.SemaphoreType.DMA,
            pltpu.SemaphoreType.DMA,
        ],
    )(idx_bm, table)


def kernel(x, table):
    batch, seq = x.shape
    idx_bm = x.astype(jnp.int32)
    out5 = _emb_lookup(idx_bm, table, batch, seq)
    out = jnp.transpose(out5, (2, 4, 0, 1, 3))
    return jnp.reshape(out, (batch, seq, EMB))


# speedup vs baseline: 2.5219x; 1.3239x over previous
import functools

import jax
import jax.numpy as jnp
from jax import lax
from jax.experimental import pallas as pl
from jax.experimental.pallas import tpu as pltpu
from jax.experimental.pallas import tpu_sc as plsc

NC = 2
NS = 16
NW = NC * NS

EMB = 64
BLK = 128
L = 16


TP = 129


def _transpose_rows_to_tiles(src, dst, rows, cols):
    lanes = lax.iota(jnp.int32, L)
    G = 8

    def col_chunk(ci, carry):
        del carry
        c0 = ci * L
        cl = c0 + lanes
        hi = lax.shift_right_logical(cl, 3)
        lo = lax.bitwise_and(cl, 7)

        def store(r, v):
            plsc.store_scatter(dst, [hi, lo, jnp.full((L,), r, jnp.int32)], v)

        vs = [src[r, pl.ds(c0, L)] for r in range(G)]
        for r0 in range(G, rows, G):
            for k in range(G):
                store(r0 - G + k, vs[k])
                vs[k] = src[r0 + k, pl.ds(c0, L)]
        for k in range(G):
            store(rows - G + k, vs[k])
        return 0

    lax.fori_loop(0, cols // L, col_chunk, 0)


NBUF = 4


def _emb_body(idx_hbm, table_hbm, out_hbm, idxa, idxt, *bufs_and_sems, seq):
    wid = lax.axis_index("s") * NC + lax.axis_index("c")
    base = wid * BLK

    pltpu.sync_copy(idx_hbm.at[pl.ds(base, BLK)], idxa)
    lanes = lax.iota(jnp.int32, L)
    for s in range(seq):
        for r0 in range(0, BLK, L):
            v = plsc.load_gather(idxa, [lanes + r0, jnp.full((L,), s, jnp.int32)])
            idxt[s, pl.ds(r0, L)] = v

    g = bufs_and_sems[:NBUF]
    t = bufs_and_sems[NBUF:2 * NBUF]
    gsems = bufs_and_sems[2 * NBUF:3 * NBUF]
    osems = bufs_and_sems[3 * NBUF:4 * NBUF]

    def gather_descr(s, b):
        return pltpu.make_async_copy(table_hbm.at[idxt.at[s]], g[b], gsems[b])

    def out_descr(s, b):
        return pltpu.make_async_copy(t[b].at[:, :, pl.ds(0, BLK)],
                                     out_hbm.at[s, :, wid], osems[b])

    for b in range(NBUF):
        gather_descr(b, b).start()

    def loop_body(i, carry):
        del carry
        for b in range(NBUF):
            s = NBUF * i + b

            @pl.when(s >= NBUF)
            def _():
                out_descr(s - NBUF, b).wait()

            gather_descr(s, b).wait()
            _transpose_rows_to_tiles(g[b], t[b], BLK, EMB)
            out_descr(s, b).start()

            @pl.when(s + NBUF < seq)
            def _():
                gather_descr(s + NBUF, b).start()

        return 0

    lax.fori_loop(0, seq // NBUF, loop_body, 0)

    for b in range(NBUF):
        out_descr(seq - NBUF + b, b).wait()


def _emb_lookup(idx_bm, table, batch, seq):
    mesh = plsc.VectorSubcoreMesh(core_axis_name="c", subcore_axis_name="s")
    body = functools.partial(_emb_body, seq=seq)
    return pl.kernel(
        body,
        out_type=jax.ShapeDtypeStruct((seq, EMB // 8, batch // BLK, 8, BLK),
                                      jnp.float32),
        mesh=mesh,
        compiler_params=pltpu.CompilerParams(use_tc_tiling_on_sc=False,
                                             needs_layout_passes=False),
        scratch_types=(
            [pltpu.VMEM((BLK, seq), jnp.int32),
             pltpu.VMEM((seq, BLK), jnp.int32)]
            + [pltpu.VMEM((BLK, EMB), jnp.float32)] * NBUF
            + [pltpu.VMEM((EMB // 8, 8, TP), jnp.float32)] * NBUF
            + [pltpu.SemaphoreType.DMA] * (2 * NBUF)
        ),
    )(idx_bm, table)


def kernel(x, table):
    batch, seq = x.shape
    idx_bm = x.astype(jnp.int32)
    out5 = _emb_lookup(idx_bm, table, batch, seq)
    out = jnp.transpose(out5, (2, 4, 0, 1, 3))
    return jnp.reshape(out, (batch, seq, EMB))
